# Initial kernel scaffold; baseline (speedup 1.0000x reference)
import jax
import jax.numpy as jnp
from jax.experimental import pallas as pl


def kernel(inputs, tfidf_svd_user_feed, tfidf_svd_feed_user, tfidf_svd_user_author, tfidf_svd_author_user, tfidf_svd_feed_emb, tfidf_svd_tag_user, tfidf_svd_hkey_user, tfidf_svd_mkey_user, tfidf_svd_tag_feed, tfidf_svd_hkey_feed, tfidf_svd_mkey_feed, user_feed_d2v, feed_user_d2v, user_author_d2v, author_user_d2v, first_order_shifts, W256, b256, W128, b128):
    raise NotImplementedError("write your pallas kernel here")


# speedup vs baseline: 2.8066x; 2.8066x over previous
import functools

import jax
import jax.numpy as jnp
import numpy as np
from jax import lax
from jax.experimental import pallas as pl
from jax.experimental.pallas import tpu as pltpu
from jax.experimental.pallas import tpu_sc as plsc

B = 1024
NC, NS, L = 2, 16, 16
NW = NC * NS
BPW = B // NW

_TABLE_DIMS = (150, 150, 32, 32, 32, 64, 64,
               150, 64, 32, 32, 32, 64,
               150, 64)
_TABLE_IDX = (0, 0, 0, 0, 0, 0, 0,
              1, 1, 1, 1, 1, 1,
              2, 2)


def _sc_gather_body(u_hbm, f_hbm, a_hbm, *rest):
    tables = rest[:15]
    outs = rest[15:30]
    u_v, f_v, a_v = rest[30:33]
    bufs = rest[33:48]
    sem = rest[48]
    idx_vs = (u_v, f_v, a_v)

    wid = lax.axis_index("s") * NC + lax.axis_index("c")
    base = wid * BPW
    pltpu.sync_copy(u_hbm.at[pl.ds(base, BPW)], u_v.at[pl.ds(0, BPW)])
    pltpu.sync_copy(f_hbm.at[pl.ds(base, BPW)], f_v.at[pl.ds(0, BPW)])
    pltpu.sync_copy(a_hbm.at[pl.ds(base, BPW)], a_v.at[pl.ds(0, BPW)])

    def step(i, _):
        rows = [idx_vs[c][pl.ds(i, 16)][0] for c in range(3)]
        for t in range(15):
            r = rows[_TABLE_IDX[t]]
            pltpu.async_copy(tables[t].at[pl.ds(r, 1), :],
                             bufs[t].at[pl.ds(i, 1), :], sem)
        return 0

    lax.fori_loop(0, BPW, step, 0)
    for t in range(15):
        pltpu.make_async_copy(tables[t].at[pl.ds(0, BPW), :], bufs[t],
                              sem).wait()
    for t in range(15):
        pltpu.sync_copy(bufs[t], outs[t].at[pl.ds(base, BPW)])


def _sc_gather(u, f, a, *tables):
    fn = pl.kernel(
        _sc_gather_body,
        mesh=plsc.VectorSubcoreMesh(core_axis_name="c", subcore_axis_name="s"),
        out_type=[jax.ShapeDtypeStruct((B, d), jnp.float32) for d in _TABLE_DIMS],
        scratch_types=(
            [pltpu.VMEM((BPW + 16,), jnp.int32) for _ in range(3)]
            + [pltpu.VMEM((BPW, d), jnp.float32) for d in _TABLE_DIMS]
            + [pltpu.SemaphoreType.DMA]
        ),
    )
    return fn(u, f, a, *tables)


_X256_OFF = 1112
_X128_OFF = 1368
_NCOLS = 1496


_RB = 256
_G = 5


def _dense_body(ue, ua, tagu, k1u, k2u, ud, uad, fe, tagf, k1f, femb, k2f,
                fd, au, aud, w256, b256, w128, b128, out):
    ue_v, ua_v = ue[...], ua[...]
    fe_v, au_v = fe[...], au[...]
    halves = ((ue_v.astype(jnp.bfloat16), fe_v.astype(jnp.bfloat16)),
              (ua_v.astype(jnp.bfloat16), au_v.astype(jnp.bfloat16)))
    acc = jnp.zeros((_RB, 256), jnp.float32)
    for g in range(300 // _G):
        uu, vv = halves[0] if g < 150 // _G else halves[1]
        i0 = (g * _G) % 150
        x = jnp.concatenate(
            [uu[:, i:i + 1] * vv for i in range(i0, i0 + _G)], axis=1)
        acc += jnp.dot(x, w256[g * 150 * _G:(g + 1) * 150 * _G, :],
                       preferred_element_type=jnp.float32)
    x256 = jnp.maximum(acc + b256[...], 0.0)

    d2v = ((ud[...], fd[...]), (uad[...], aud[...]))
    a2 = jnp.zeros((_RB, 128), jnp.float32)
    for h, (uu32, vv32) in enumerate(d2v):
        uu = uu32.astype(jnp.bfloat16)
        vv = vv32.astype(jnp.bfloat16)
        for g in range(8):
            xp = jnp.concatenate(
                [uu[:, i:i + 1] * vv for i in range(8 * g, 8 * g + 8)],
                axis=1)
            a2 = a2 + jnp.dot(
                xp, w128[h * 4096 + g * 512:h * 4096 + (g + 1) * 512, :],
                preferred_element_type=jnp.float32)
    x128 = jnp.maximum(a2 + b128[...], 0.0)

    out[:, 0:150] = ue_v
    out[:, 150:300] = ua_v
    out[:, 300:332] = tagu[...]
    out[:, 332:364] = k1u[...]
    out[:, 364:396] = k2u[...]
    out[:, 396:460] = d2v[0][0]
    out[:, 460:524] = d2v[1][0]
    out[:, 524:674] = fe_v
    out[:, 674:706] = tagf[...]
    out[:, 706:738] = k1f[...]
    out[:, 738:802] = femb[...]
    out[:, 802:834] = k2f[...]
    out[:, 834:898] = d2v[0][1]
    out[:, 898:1048] = au_v
    out[:, 1048:1112] = d2v[1][1]
    out[:, _X256_OFF:_X128_OFF] = x256
    out[:, _X128_OFF:_NCOLS] = x128


def _row_spec(d):
    return pl.BlockSpec((_RB, d), lambda k: (k, 0))


def _const_spec(shape):
    return pl.BlockSpec(shape, lambda k: tuple(0 for _ in shape))


def _tc_in_specs():
    return [
        _row_spec(150), _row_spec(150),
        _row_spec(32), _row_spec(32), _row_spec(32),
        _row_spec(64), _row_spec(64),
        _row_spec(150),
        _row_spec(32), _row_spec(32),
        _row_spec(64), _row_spec(32), _row_spec(64),
        _row_spec(150), _row_spec(64),
        _const_spec((45000, 256)),
        _const_spec((1, 256)),
        _const_spec((8192, 128)),
        _const_spec((1, 128)),
    ]


def _tc_dense(ue, ua, tagu, k1u, k2u, ud, uad, fe, tagf, k1f, femb, k2f,
              fd, au, aud, w256, b256, w128, b128):
    return pl.pallas_call(
        _dense_body,
        grid=(B // _RB,),
        in_specs=_tc_in_specs(),
        out_specs=pl.BlockSpec((_RB, _NCOLS), lambda k: (k, 0)),
        out_shape=jax.ShapeDtypeStruct((B, _NCOLS), jnp.float32),
    )(ue, ua, tagu, k1u, k2u, ud, uad, fe, tagf, k1f, femb, k2f, fd, au, aud,
      w256, b256, w128, b128)


def kernel(inputs, tfidf_svd_user_feed, tfidf_svd_feed_user,
           tfidf_svd_user_author, tfidf_svd_author_user, tfidf_svd_feed_emb,
           tfidf_svd_tag_user, tfidf_svd_hkey_user, tfidf_svd_mkey_user,
           tfidf_svd_tag_feed, tfidf_svd_hkey_feed, tfidf_svd_mkey_feed,
           user_feed_d2v, feed_user_d2v, user_author_d2v, author_user_d2v,
           first_order_shifts, W256, b256, W128, b128):
    inputs = inputs.reshape(-1, 3)
    one_hot_inputs = inputs + first_order_shifts[None, :]
    u = inputs[:, 0]
    f = inputs[:, 1]
    a = inputs[:, 2]
    (ue, ua, tagu, k1u, k2u, ud, uad,
     fe, femb, tagf, k1f, k2f, fd,
     au, aud) = _sc_gather(
        u, f, a,
        tfidf_svd_user_feed, tfidf_svd_user_author, tfidf_svd_tag_user,
        tfidf_svd_hkey_user, tfidf_svd_mkey_user, user_feed_d2v,
        user_author_d2v,
        tfidf_svd_feed_user, tfidf_svd_feed_emb, tfidf_svd_tag_feed,
        tfidf_svd_hkey_feed, tfidf_svd_mkey_feed, feed_user_d2v,
        tfidf_svd_author_user, author_user_d2v)
    expert = _tc_dense(ue, ua, tagu, k1u, k2u, ud, uad, fe, tagf, k1f, femb,
                       k2f, fd, au, aud, W256.astype(jnp.bfloat16),
                       b256.reshape(1, -1), W128.astype(jnp.bfloat16),
                       b128.reshape(1, -1))
    embed_inputs = jnp.stack([ue, ua, fe, au], axis=0).reshape(-1, 4, 150)
    return (expert, one_hot_inputs, embed_inputs)
